# baseline (speedup 1.0000x reference)
import functools

import jax
import jax.numpy as jnp
from jax import lax
from jax.experimental import pallas as pl
from jax.experimental.pallas import tpu as pltpu
from jax.experimental.pallas import tpu_sc as plsc

N_NODES = 10000
N_EDGES = 160000
D_IN = 256
D_HID = 256
N_CLASSES = 128

NC = 2
NS = 16
L = 16

N_PAD = 10240
ROWS_PER_TILE = N_PAD // NS
B_EDGE = 64
NB_A = 58
NB_B = 100
NB_MAX = max(NB_A, NB_B)
E_PAD = NS * (NB_A + NB_B) * B_EDGE
EPT32 = E_PAD // (NC * NS)

_mesh = plsc.VectorSubcoreMesh(core_axis_name="c", subcore_axis_name="s")
_sc_params = pltpu.CompilerParams(needs_layout_passes=False,
                                  use_tc_tiling_on_sc=False)


@functools.partial(
    pl.kernel,
    out_type=(
        jax.ShapeDtypeStruct((NC * NS, N_PAD), jnp.float32),
        jax.ShapeDtypeStruct((NC * NS, N_PAD), jnp.float32),
    ),
    mesh=_mesh,
    scratch_types=[
        pltpu.VMEM((EPT32,), jnp.int32),
        pltpu.VMEM((EPT32,), jnp.int32),
        pltpu.VMEM((N_PAD,), jnp.float32),
        pltpu.VMEM((N_PAD,), jnp.float32),
    ],
    compiler_params=_sc_params,
)
def _deg_kernel(src_hbm, dst_hbm, odeg_hbm, ideg_hbm, src_v, dst_v, oacc, iacc):
    wid = lax.axis_index("s") * NC + lax.axis_index("c")
    base = wid * EPT32
    pltpu.sync_copy(src_hbm.at[pl.ds(base, EPT32)], src_v)
    pltpu.sync_copy(dst_hbm.at[pl.ds(base, EPT32)], dst_v)

    zeros16 = jnp.zeros((L,), jnp.float32)

    def zero_body(i, _):
        oacc[pl.ds(i * L, L)] = zeros16
        iacc[pl.ds(i * L, L)] = zeros16
        return 0

    lax.fori_loop(0, N_PAD // L, zero_body, 0)

    ones16 = jnp.ones((L,), jnp.float32)

    def edge_body(i, _):
        s16 = src_v[pl.ds(i * L, L)]
        d16 = dst_v[pl.ds(i * L, L)]
        plsc.addupdate_scatter(oacc, [s16], ones16)
        plsc.addupdate_scatter(iacc, [d16], ones16)
        return 0

    lax.fori_loop(0, EPT32 // L, edge_body, 0)

    pltpu.sync_copy(oacc, odeg_hbm.at[wid])
    pltpu.sync_copy(iacc, ideg_hbm.at[wid])


def _make_agg_kernel(dc):
    @functools.partial(
        pl.kernel,
        out_type=(
            jax.ShapeDtypeStruct((N_PAD, dc), jnp.bfloat16),
            jax.ShapeDtypeStruct((N_PAD, dc), jnp.bfloat16),
        ),
        mesh=_mesh,
        scratch_types=[
            pltpu.VMEM((NB_MAX, B_EDGE), jnp.int32),
            pltpu.VMEM((NB_MAX, B_EDGE), jnp.int32),
            pltpu.VMEM((3, B_EDGE, dc), jnp.bfloat16),
            pltpu.VMEM_SHARED((N_PAD, dc), jnp.bfloat16),
            pltpu.SemaphoreType.DMA,
            pltpu.SemaphoreType.DMA,
        ],
        compiler_params=_sc_params,
    )
    def agg(ha_hbm, hb_hbm, srca_hbm, dsta_hbm, srcb_hbm, dstb_hbm,
            zeros_hbm, outa_hbm, outb_hbm,
            src_v, dst_v, rows_v, acc_sh, gsem, ssem):
        cid = lax.axis_index("c")
        sid = lax.axis_index("s")
        row0 = sid * ROWS_PER_TILE

        @pl.when(cid == 0)
        def _():
            pltpu.sync_copy(srca_hbm.at[sid], src_v.at[pl.ds(0, NB_A)])
            pltpu.sync_copy(dsta_hbm.at[sid], dst_v.at[pl.ds(0, NB_A)])

        @pl.when(cid == 1)
        def _():
            pltpu.sync_copy(srcb_hbm.at[sid], src_v.at[pl.ds(0, NB_B)])
            pltpu.sync_copy(dstb_hbm.at[sid], dst_v.at[pl.ds(0, NB_B)])

        pltpu.sync_copy(zeros_hbm, acc_sh.at[pl.ds(row0, ROWS_PER_TILE)])
        plsc.subcore_barrier()

        def run(h_hbm, nb):
            pltpu.async_copy(h_hbm.at[src_v.at[0]], rows_v.at[0], gsem)
            pltpu.async_copy(h_hbm.at[src_v.at[1]], rows_v.at[1], gsem)

            def body(j, _):
                slot = lax.rem(j, 3)
                pltpu.make_async_copy(h_hbm.at[src_v.at[j]], rows_v.at[slot],
                                      gsem).wait()

                @pl.when(j >= 1)
                def _():
                    jp = j - 1
                    pltpu.make_async_copy(rows_v.at[lax.rem(jp, 3)],
                                          acc_sh.at[dst_v.at[jp]], ssem).wait()

                @pl.when(j + 2 < nb)
                def _():
                    pltpu.async_copy(h_hbm.at[src_v.at[j + 2]],
                                     rows_v.at[lax.rem(j + 2, 3)], gsem)

                pltpu.async_copy(rows_v.at[slot], acc_sh.at[dst_v.at[j]],
                                 ssem, add=True)
                return 0

            lax.fori_loop(0, nb, body, 0)
            jl = nb - 1
            pltpu.make_async_copy(rows_v.at[lax.rem(jl, 3)],
                                  acc_sh.at[dst_v.at[jl]], ssem).wait()

        @pl.when(cid == 0)
        def _():
            run(ha_hbm, NB_A)

        @pl.when(cid == 1)
        def _():
            run(hb_hbm, NB_B)

        plsc.subcore_barrier()

        @pl.when(cid == 0)
        def _():
            pltpu.sync_copy(acc_sh.at[pl.ds(row0, ROWS_PER_TILE)],
                            outa_hbm.at[pl.ds(row0, ROWS_PER_TILE)])

        @pl.when(cid == 1)
        def _():
            pltpu.sync_copy(acc_sh.at[pl.ds(row0, ROWS_PER_TILE)],
                            outb_hbm.at[pl.ds(row0, ROWS_PER_TILE)])

    return agg


_agg256 = _make_agg_kernel(256)
_agg128p = _make_agg_kernel(128)


_ROW_BLK = 256
_N_BLKS = N_PAD // _ROW_BLK


def _scales(odeg_blk, ideg_blk):
    odeg = jnp.sum(odeg_blk, axis=0)
    ideg = jnp.sum(ideg_blk, axis=0)
    return (lax.rsqrt(jnp.maximum(odeg, 1.0)), lax.rsqrt(jnp.maximum(ideg, 1.0)))


def _prep_body(x_ref, w1_ref, odeg_ref, ideg_ref, h1a_ref, h1b_ref,
               sin_ref, sout_ref):
    s_out, s_in = _scales(odeg_ref[...], ideg_ref[...])
    sin_ref[...] = s_in[:, None]
    sout_ref[...] = s_out[:, None]
    h = jnp.dot(x_ref[...] * s_out[:, None], w1_ref[...],
                preferred_element_type=jnp.float32).astype(jnp.bfloat16)
    h1a_ref[...] = h
    h1b_ref[...] = h


def _prep_call(x_pad, w1, odeg_p, ideg_p):
    return pl.pallas_call(
        _prep_body,
        grid=(_N_BLKS,),
        in_specs=[
            pl.BlockSpec((_ROW_BLK, D_IN), lambda i: (i, 0)),
            pl.BlockSpec((D_IN, D_HID), lambda i: (0, 0)),
            pl.BlockSpec((NC * NS, _ROW_BLK), lambda i: (0, i)),
            pl.BlockSpec((NC * NS, _ROW_BLK), lambda i: (0, i)),
        ],
        out_specs=(
            pl.BlockSpec((_ROW_BLK, D_HID), lambda i: (i, 0)),
            pl.BlockSpec((_ROW_BLK, D_HID), lambda i: (i, 0)),
            pl.BlockSpec((_ROW_BLK, 1), lambda i: (i, 0)),
            pl.BlockSpec((_ROW_BLK, 1), lambda i: (i, 0)),
        ),
        out_shape=(
            jax.ShapeDtypeStruct((N_PAD, D_HID), jnp.bfloat16),
            jax.ShapeDtypeStruct((N_PAD, D_HID), jnp.bfloat16),
            jax.ShapeDtypeStruct((N_PAD, 1), jnp.float32),
            jax.ShapeDtypeStruct((N_PAD, 1), jnp.float32),
        ),
    )(x_pad, w1, odeg_p, ideg_p)


def _mid_body(a1a_ref, a1b_ref, sin_ref, sout_ref, b1_ref, w2_ref,
              h2a_ref, h2b_ref):
    agg = (a1a_ref[...].astype(jnp.float32) + a1b_ref[...].astype(jnp.float32))
    z = jnp.maximum(agg * sin_ref[...] + b1_ref[...], 0.0)
    h2 = jnp.dot(z * sout_ref[...], w2_ref[...],
                 preferred_element_type=jnp.float32).astype(jnp.bfloat16)
    h2a_ref[...] = h2
    h2b_ref[...] = h2


def _mid_call(a1a, a1b, s_in, s_out, b1, w2):
    return pl.pallas_call(
        _mid_body,
        grid=(_N_BLKS,),
        in_specs=[
            pl.BlockSpec((_ROW_BLK, D_HID), lambda i: (i, 0)),
            pl.BlockSpec((_ROW_BLK, D_HID), lambda i: (i, 0)),
            pl.BlockSpec((_ROW_BLK, 1), lambda i: (i, 0)),
            pl.BlockSpec((_ROW_BLK, 1), lambda i: (i, 0)),
            pl.BlockSpec((1, D_HID), lambda i: (0, 0)),
            pl.BlockSpec((D_HID, N_CLASSES), lambda i: (0, 0)),
        ],
        out_specs=(
            pl.BlockSpec((_ROW_BLK, N_CLASSES), lambda i: (i, 0)),
            pl.BlockSpec((_ROW_BLK, N_CLASSES), lambda i: (i, 0)),
        ),
        out_shape=(
            jax.ShapeDtypeStruct((N_PAD, N_CLASSES), jnp.bfloat16),
            jax.ShapeDtypeStruct((N_PAD, N_CLASSES), jnp.bfloat16),
        ),
    )(a1a, a1b, s_in, s_out, b1, w2)


_F_BLK = 200
_F_BLKS = N_NODES // _F_BLK


def _final_body(a2a_ref, a2b_ref, sin_ref, b2_ref, out_ref):
    agg = (a2a_ref[...].astype(jnp.float32) + a2b_ref[...].astype(jnp.float32))
    out_ref[...] = agg * sin_ref[...] + b2_ref[...]


def _final_call(a2a, a2b, s_in, b2):
    return pl.pallas_call(
        _final_body,
        grid=(_F_BLKS,),
        in_specs=[
            pl.BlockSpec((_F_BLK, N_CLASSES), lambda i: (i, 0)),
            pl.BlockSpec((_F_BLK, N_CLASSES), lambda i: (i, 0)),
            pl.BlockSpec((_F_BLK, 1), lambda i: (i, 0)),
            pl.BlockSpec((1, N_CLASSES), lambda i: (0, 0)),
        ],
        out_specs=pl.BlockSpec((_F_BLK, N_CLASSES), lambda i: (i, 0)),
        out_shape=jax.ShapeDtypeStruct((N_NODES, N_CLASSES), jnp.float32),
    )(a2a, a2b, s_in, b2)


def kernel(x, edge_index, W1, b1, W2, b2):
    src = edge_index[0]
    dst = edge_index[1]
    pad = jnp.full((E_PAD - N_EDGES,), N_NODES, dtype=jnp.int32)
    srcp = jnp.concatenate([src, pad])
    dstp = jnp.concatenate([dst, pad])
    e_a = NS * NB_A * B_EDGE
    src_a = srcp[:e_a].reshape(NS, NB_A, B_EDGE)
    dst_a = dstp[:e_a].reshape(NS, NB_A, B_EDGE)
    src_c = srcp[e_a:].reshape(NS, NB_B, B_EDGE)
    dst_c = dstp[e_a:].reshape(NS, NB_B, B_EDGE)

    x_pad = jnp.pad(x, ((0, N_PAD - N_NODES), (0, 0)))
    z256 = jnp.zeros((ROWS_PER_TILE, 256), jnp.bfloat16)
    z128 = jnp.zeros((ROWS_PER_TILE, 128), jnp.bfloat16)

    odeg_p, ideg_p = _deg_kernel(srcp, dstp)

    h1a, h1b, s_in, s_out = _prep_call(x_pad, W1, odeg_p, ideg_p)
    a1a, a1b = _agg256(h1a, h1b, src_a, dst_a, src_c, dst_c, z256)
    h2a, h2b = _mid_call(a1a, a1b, s_in, s_out, b1.reshape(1, -1), W2)
    a2a, a2b = _agg128p(h2a, h2b, src_a, dst_a, src_c, dst_c, z128)
    return _final_call(a2a, a2b, s_in, b2.reshape(1, -1))


# speedup vs baseline: 1.2113x; 1.2113x over previous
import functools

import jax
import jax.numpy as jnp
from jax import lax
from jax.experimental import pallas as pl
from jax.experimental.pallas import tpu as pltpu
from jax.experimental.pallas import tpu_sc as plsc

N_NODES = 10000
N_EDGES = 160000
D_IN = 256
D_HID = 256
N_CLASSES = 128

NC = 2
NS = 16
L = 16

N_PAD = 10240
ROWS_PER_TILE = N_PAD // NS
B_EDGE = 64
NB = 158
E_PAD = NS * NB * B_EDGE
EPT32 = E_PAD // (NC * NS)

_mesh = plsc.VectorSubcoreMesh(core_axis_name="c", subcore_axis_name="s")
_sc_params = pltpu.CompilerParams(needs_layout_passes=False,
                                  use_tc_tiling_on_sc=False)


@functools.partial(
    pl.kernel,
    out_type=(
        jax.ShapeDtypeStruct((NC * NS, N_PAD), jnp.float32),
        jax.ShapeDtypeStruct((NC * NS, N_PAD), jnp.float32),
    ),
    mesh=_mesh,
    scratch_types=[
        pltpu.VMEM((EPT32,), jnp.int32),
        pltpu.VMEM((EPT32,), jnp.int32),
        pltpu.VMEM((N_PAD,), jnp.float32),
        pltpu.VMEM((N_PAD,), jnp.float32),
    ],
    compiler_params=_sc_params,
)
def _deg_kernel(src_hbm, dst_hbm, odeg_hbm, ideg_hbm, src_v, dst_v, oacc, iacc):
    wid = lax.axis_index("s") * NC + lax.axis_index("c")
    base = wid * EPT32
    pltpu.sync_copy(src_hbm.at[pl.ds(base, EPT32)], src_v)
    pltpu.sync_copy(dst_hbm.at[pl.ds(base, EPT32)], dst_v)

    zeros16 = jnp.zeros((L,), jnp.float32)

    def zero_body(i, _):
        oacc[pl.ds(i * L, L)] = zeros16
        iacc[pl.ds(i * L, L)] = zeros16
        return 0

    lax.fori_loop(0, N_PAD // L, zero_body, 0)

    ones16 = jnp.ones((L,), jnp.float32)

    def edge_body(i, _):
        s16 = src_v[pl.ds(i * L, L)]
        d16 = dst_v[pl.ds(i * L, L)]
        plsc.addupdate_scatter(oacc, [s16], ones16)
        plsc.addupdate_scatter(iacc, [d16], ones16)
        return 0

    lax.fori_loop(0, EPT32 // L, edge_body, 0)

    pltpu.sync_copy(oacc, odeg_hbm.at[wid])
    pltpu.sync_copy(iacc, ideg_hbm.at[wid])


def _make_agg_kernel(dc):
    @functools.partial(
        pl.kernel,
        out_type=(
            jax.ShapeDtypeStruct((N_PAD, dc), jnp.bfloat16),
            jax.ShapeDtypeStruct((N_PAD, dc), jnp.bfloat16),
        ),
        mesh=_mesh,
        scratch_types=[
            pltpu.VMEM((NB, B_EDGE), jnp.int32),
            pltpu.VMEM((NB, B_EDGE), jnp.int32),
            pltpu.VMEM((3, B_EDGE, dc), jnp.bfloat16),
            pltpu.VMEM_SHARED((N_PAD, dc), jnp.bfloat16),
            pltpu.SemaphoreType.DMA,
            pltpu.SemaphoreType.DMA,
        ],
        compiler_params=_sc_params,
    )
    def agg(ha_hbm, hb_hbm, src_hbm, dst_hbm, zeros_hbm, outa_hbm, outb_hbm,
            src_v, dst_v, rows_v, acc_sh, gsem, ssem):
        cid = lax.axis_index("c")
        sid = lax.axis_index("s")
        row0 = sid * ROWS_PER_TILE

        pltpu.sync_copy(src_hbm.at[sid], src_v)
        pltpu.sync_copy(dst_hbm.at[sid], dst_v)
        pltpu.sync_copy(zeros_hbm, acc_sh.at[pl.ds(row0, ROWS_PER_TILE)])
        plsc.subcore_barrier()

        def run(h_hbm, nb):
            pltpu.async_copy(h_hbm.at[src_v.at[0]], rows_v.at[0], gsem)
            pltpu.async_copy(h_hbm.at[src_v.at[1]], rows_v.at[1], gsem)

            def body(j, _):
                slot = lax.rem(j, 3)
                pltpu.make_async_copy(h_hbm.at[src_v.at[j]], rows_v.at[slot],
                                      gsem).wait()

                @pl.when(j >= 1)
                def _():
                    jp = j - 1
                    pltpu.make_async_copy(rows_v.at[lax.rem(jp, 3)],
                                          acc_sh.at[dst_v.at[jp]], ssem).wait()

                @pl.when(j + 2 < nb)
                def _():
                    pltpu.async_copy(h_hbm.at[src_v.at[j + 2]],
                                     rows_v.at[lax.rem(j + 2, 3)], gsem)

                pltpu.async_copy(rows_v.at[slot], acc_sh.at[dst_v.at[j]],
                                 ssem, add=True)
                return 0

            lax.fori_loop(0, nb, body, 0)
            jl = nb - 1
            pltpu.make_async_copy(rows_v.at[lax.rem(jl, 3)],
                                  acc_sh.at[dst_v.at[jl]], ssem).wait()

        @pl.when(cid == 0)
        def _():
            run(ha_hbm, NB)

        @pl.when(cid == 1)
        def _():
            run(hb_hbm, NB)

        plsc.subcore_barrier()

        @pl.when(cid == 0)
        def _():
            pltpu.sync_copy(acc_sh.at[pl.ds(row0, ROWS_PER_TILE)],
                            outa_hbm.at[pl.ds(row0, ROWS_PER_TILE)])

        @pl.when(cid == 1)
        def _():
            pltpu.sync_copy(acc_sh.at[pl.ds(row0, ROWS_PER_TILE)],
                            outb_hbm.at[pl.ds(row0, ROWS_PER_TILE)])

    return agg


_agg128 = _make_agg_kernel(128)
_agg64 = _make_agg_kernel(64)


_ROW_BLK = 256
_N_BLKS = N_PAD // _ROW_BLK


def _scales(odeg_blk, ideg_blk):
    odeg = jnp.sum(odeg_blk, axis=0)
    ideg = jnp.sum(ideg_blk, axis=0)
    return (lax.rsqrt(jnp.maximum(odeg, 1.0)), lax.rsqrt(jnp.maximum(ideg, 1.0)))


def _prep_body(x_ref, w1_ref, odeg_ref, ideg_ref, h1a_ref, h1b_ref,
               sin_ref, sout_ref):
    s_out, s_in = _scales(odeg_ref[...], ideg_ref[...])
    sin_ref[...] = s_in[:, None]
    sout_ref[...] = s_out[:, None]
    h = jnp.dot(x_ref[...] * s_out[:, None], w1_ref[...],
                preferred_element_type=jnp.float32).astype(jnp.bfloat16)
    h1a_ref[...] = h[:, :D_HID // 2]
    h1b_ref[...] = h[:, D_HID // 2:]


def _prep_call(x_pad, w1, odeg_p, ideg_p):
    return pl.pallas_call(
        _prep_body,
        grid=(_N_BLKS,),
        in_specs=[
            pl.BlockSpec((_ROW_BLK, D_IN), lambda i: (i, 0)),
            pl.BlockSpec((D_IN, D_HID), lambda i: (0, 0)),
            pl.BlockSpec((NC * NS, _ROW_BLK), lambda i: (0, i)),
            pl.BlockSpec((NC * NS, _ROW_BLK), lambda i: (0, i)),
        ],
        out_specs=(
            pl.BlockSpec((_ROW_BLK, D_HID // 2), lambda i: (i, 0)),
            pl.BlockSpec((_ROW_BLK, D_HID // 2), lambda i: (i, 0)),
            pl.BlockSpec((_ROW_BLK, 1), lambda i: (i, 0)),
            pl.BlockSpec((_ROW_BLK, 1), lambda i: (i, 0)),
        ),
        out_shape=(
            jax.ShapeDtypeStruct((N_PAD, D_HID // 2), jnp.bfloat16),
            jax.ShapeDtypeStruct((N_PAD, D_HID // 2), jnp.bfloat16),
            jax.ShapeDtypeStruct((N_PAD, 1), jnp.float32),
            jax.ShapeDtypeStruct((N_PAD, 1), jnp.float32),
        ),
    )(x_pad, w1, odeg_p, ideg_p)


def _mid_body(a1a_ref, a1b_ref, sin_ref, sout_ref, b1_ref, w2_ref,
              h2a_ref, h2b_ref):
    agg = jnp.concatenate(
        [a1a_ref[...].astype(jnp.float32), a1b_ref[...].astype(jnp.float32)],
        axis=1)
    z = jnp.maximum(agg * sin_ref[...] + b1_ref[...], 0.0)
    h2 = jnp.dot(z * sout_ref[...], w2_ref[...],
                 preferred_element_type=jnp.float32).astype(jnp.bfloat16)
    h2a_ref[...] = h2[:, :N_CLASSES // 2]
    h2b_ref[...] = h2[:, N_CLASSES // 2:]


def _mid_call(a1a, a1b, s_in, s_out, b1, w2):
    return pl.pallas_call(
        _mid_body,
        grid=(_N_BLKS,),
        in_specs=[
            pl.BlockSpec((_ROW_BLK, D_HID // 2), lambda i: (i, 0)),
            pl.BlockSpec((_ROW_BLK, D_HID // 2), lambda i: (i, 0)),
            pl.BlockSpec((_ROW_BLK, 1), lambda i: (i, 0)),
            pl.BlockSpec((_ROW_BLK, 1), lambda i: (i, 0)),
            pl.BlockSpec((1, D_HID), lambda i: (0, 0)),
            pl.BlockSpec((D_HID, N_CLASSES), lambda i: (0, 0)),
        ],
        out_specs=(
            pl.BlockSpec((_ROW_BLK, N_CLASSES // 2), lambda i: (i, 0)),
            pl.BlockSpec((_ROW_BLK, N_CLASSES // 2), lambda i: (i, 0)),
        ),
        out_shape=(
            jax.ShapeDtypeStruct((N_PAD, N_CLASSES // 2), jnp.bfloat16),
            jax.ShapeDtypeStruct((N_PAD, N_CLASSES // 2), jnp.bfloat16),
        ),
    )(a1a, a1b, s_in, s_out, b1, w2)


_F_BLK = 200
_F_BLKS = N_NODES // _F_BLK


def _final_body(a2a_ref, a2b_ref, sin_ref, b2_ref, out_ref):
    agg = jnp.concatenate(
        [a2a_ref[...].astype(jnp.float32), a2b_ref[...].astype(jnp.float32)],
        axis=1)
    out_ref[...] = agg * sin_ref[...] + b2_ref[...]


def _final_call(a2a, a2b, s_in, b2):
    return pl.pallas_call(
        _final_body,
        grid=(_F_BLKS,),
        in_specs=[
            pl.BlockSpec((_F_BLK, N_CLASSES // 2), lambda i: (i, 0)),
            pl.BlockSpec((_F_BLK, N_CLASSES // 2), lambda i: (i, 0)),
            pl.BlockSpec((_F_BLK, 1), lambda i: (i, 0)),
            pl.BlockSpec((1, N_CLASSES), lambda i: (0, 0)),
        ],
        out_specs=pl.BlockSpec((_F_BLK, N_CLASSES), lambda i: (i, 0)),
        out_shape=jax.ShapeDtypeStruct((N_NODES, N_CLASSES), jnp.float32),
    )(a2a, a2b, s_in, b2)


def kernel(x, edge_index, W1, b1, W2, b2):
    src = edge_index[0]
    dst = edge_index[1]
    pad = jnp.full((E_PAD - N_EDGES,), N_NODES, dtype=jnp.int32)
    srcp = jnp.concatenate([src, pad])
    dstp = jnp.concatenate([dst, pad])
    src_b = srcp.reshape(NS, NB, B_EDGE)
    dst_b = dstp.reshape(NS, NB, B_EDGE)

    x_pad = jnp.pad(x, ((0, N_PAD - N_NODES), (0, 0)))
    z128 = jnp.zeros((ROWS_PER_TILE, 128), jnp.bfloat16)
    z64 = jnp.zeros((ROWS_PER_TILE, 64), jnp.bfloat16)

    odeg_p, ideg_p = _deg_kernel(srcp, dstp)

    h1a, h1b, s_in, s_out = _prep_call(x_pad, W1, odeg_p, ideg_p)
    a1a, a1b = _agg128(h1a, h1b, src_b, dst_b, z128)
    h2a, h2b = _mid_call(a1a, a1b, s_in, s_out, b1.reshape(1, -1), W2)
    a2a, a2b = _agg64(h2a, h2b, src_b, dst_b, z64)
    return _final_call(a2a, a2b, s_in, b2.reshape(1, -1))


# speedup vs baseline: 1.3483x; 1.1131x over previous
import functools

import jax
import jax.numpy as jnp
from jax import lax
from jax.experimental import pallas as pl
from jax.experimental.pallas import tpu as pltpu
from jax.experimental.pallas import tpu_sc as plsc

N_NODES = 10000
N_EDGES = 160000
D_IN = 256
D_HID = 256
N_CLASSES = 128

NC = 2
NS = 16
L = 16

N_PAD = 10240
ROWS_PER_TILE = N_PAD // NS
B_EDGE = 64
NB = 158
E_PAD = NS * NB * B_EDGE
EPT32 = E_PAD // (NC * NS)

_mesh = plsc.VectorSubcoreMesh(core_axis_name="c", subcore_axis_name="s")
_sc_params = pltpu.CompilerParams(needs_layout_passes=False,
                                  use_tc_tiling_on_sc=False)


@functools.partial(
    pl.kernel,
    out_type=(
        jax.ShapeDtypeStruct((NC * NS, N_PAD), jnp.float32),
        jax.ShapeDtypeStruct((NC * NS, N_PAD), jnp.float32),
    ),
    mesh=_mesh,
    scratch_types=[
        pltpu.VMEM((EPT32,), jnp.int32),
        pltpu.VMEM((EPT32,), jnp.int32),
        pltpu.VMEM((N_PAD,), jnp.float32),
        pltpu.VMEM((N_PAD,), jnp.float32),
    ],
    compiler_params=_sc_params,
)
def _deg_kernel(src_hbm, dst_hbm, odeg_hbm, ideg_hbm, src_v, dst_v, oacc, iacc):
    wid = lax.axis_index("s") * NC + lax.axis_index("c")
    base = wid * EPT32
    pltpu.sync_copy(src_hbm.at[pl.ds(base, EPT32)], src_v)
    pltpu.sync_copy(dst_hbm.at[pl.ds(base, EPT32)], dst_v)

    zeros16 = jnp.zeros((L,), jnp.float32)

    def zero_body(i, _):
        oacc[pl.ds(i * L, L)] = zeros16
        iacc[pl.ds(i * L, L)] = zeros16
        return 0

    lax.fori_loop(0, N_PAD // L, zero_body, 0)

    ones16 = jnp.ones((L,), jnp.float32)

    def edge_body(i, _):
        s16 = src_v[pl.ds(i * L, L)]
        d16 = dst_v[pl.ds(i * L, L)]
        plsc.addupdate_scatter(oacc, [s16], ones16)
        plsc.addupdate_scatter(iacc, [d16], ones16)
        return 0

    lax.fori_loop(0, EPT32 // L, edge_body, 0)

    pltpu.sync_copy(oacc, odeg_hbm.at[wid])
    pltpu.sync_copy(iacc, ideg_hbm.at[wid])


def _make_agg_kernel(dc):
    @functools.partial(
        pl.kernel,
        out_type=(
            jax.ShapeDtypeStruct((N_PAD, dc), jnp.bfloat16),
            jax.ShapeDtypeStruct((N_PAD, dc), jnp.bfloat16),
        ),
        mesh=_mesh,
        scratch_types=[
            pltpu.VMEM((NB, B_EDGE), jnp.int32),
            pltpu.VMEM((NB, B_EDGE), jnp.int32),
            pltpu.VMEM((3, B_EDGE, dc), jnp.bfloat16),
            pltpu.VMEM_SHARED((N_PAD, dc), jnp.bfloat16),
            pltpu.SemaphoreType.DMA,
            pltpu.SemaphoreType.DMA,
        ],
        compiler_params=_sc_params,
    )
    def agg(ha_hbm, hb_hbm, src_hbm, dst_hbm, zeros_hbm, outa_hbm, outb_hbm,
            src_v, dst_v, rows_v, acc_sh, gsem, ssem):
        cid = lax.axis_index("c")
        sid = lax.axis_index("s")
        row0 = sid * ROWS_PER_TILE

        pltpu.sync_copy(src_hbm.at[sid], src_v)
        pltpu.sync_copy(dst_hbm.at[sid], dst_v)
        pltpu.sync_copy(zeros_hbm, acc_sh.at[pl.ds(row0, ROWS_PER_TILE)])
        plsc.subcore_barrier()

        def run(h_hbm, nb):
            pltpu.async_copy(h_hbm.at[src_v.at[0]], rows_v.at[0], gsem)
            pltpu.async_copy(h_hbm.at[src_v.at[1]], rows_v.at[1], gsem)

            def body(j, _):
                slot = lax.rem(j, 3)
                pltpu.make_async_copy(h_hbm.at[src_v.at[j]], rows_v.at[slot],
                                      gsem).wait()

                @pl.when(j >= 1)
                def _():
                    jp = j - 1
                    pltpu.make_async_copy(rows_v.at[lax.rem(jp, 3)],
                                          acc_sh.at[dst_v.at[jp]], ssem).wait()

                @pl.when(j + 2 < nb)
                def _():
                    pltpu.async_copy(h_hbm.at[src_v.at[j + 2]],
                                     rows_v.at[lax.rem(j + 2, 3)], gsem)

                pltpu.async_copy(rows_v.at[slot], acc_sh.at[dst_v.at[j]],
                                 ssem, add=True)
                return 0

            lax.fori_loop(0, nb, body, 0)
            jl = nb - 1
            pltpu.make_async_copy(rows_v.at[lax.rem(jl, 3)],
                                  acc_sh.at[dst_v.at[jl]], ssem).wait()

        @pl.when(cid == 0)
        def _():
            run(ha_hbm, NB)

        @pl.when(cid == 1)
        def _():
            run(hb_hbm, NB)

        plsc.subcore_barrier()

        @pl.when(cid == 0)
        def _():
            pltpu.sync_copy(acc_sh.at[pl.ds(row0, ROWS_PER_TILE)],
                            outa_hbm.at[pl.ds(row0, ROWS_PER_TILE)])

        @pl.when(cid == 1)
        def _():
            pltpu.sync_copy(acc_sh.at[pl.ds(row0, ROWS_PER_TILE)],
                            outb_hbm.at[pl.ds(row0, ROWS_PER_TILE)])

    return agg


_agg128 = _make_agg_kernel(128)
_agg64 = _make_agg_kernel(64)


_ROW_BLK = 512
_N_BLKS = N_PAD // _ROW_BLK


def _scales(odeg_blk, ideg_blk):
    odeg = jnp.sum(odeg_blk, axis=0)
    ideg = jnp.sum(ideg_blk, axis=0)
    return (lax.rsqrt(jnp.maximum(odeg, 1.0)), lax.rsqrt(jnp.maximum(ideg, 1.0)))


def _prep_body(x_ref, w1_ref, odeg_ref, ideg_ref, h1a_ref, h1b_ref,
               sin_ref, sout_ref):
    s_out, s_in = _scales(odeg_ref[...], ideg_ref[...])
    sin_ref[...] = s_in[:, None]
    sout_ref[...] = s_out[:, None]
    h = jnp.dot((x_ref[...] * s_out[:, None]).astype(jnp.bfloat16),
                w1_ref[...].astype(jnp.bfloat16),
                preferred_element_type=jnp.float32).astype(jnp.bfloat16)
    h1a_ref[...] = h[:, :D_HID // 2]
    h1b_ref[...] = h[:, D_HID // 2:]


def _prep_call(x_pad, w1, odeg_p, ideg_p):
    return pl.pallas_call(
        _prep_body,
        grid=(_N_BLKS,),
        in_specs=[
            pl.BlockSpec((_ROW_BLK, D_IN), lambda i: (i, 0)),
            pl.BlockSpec((D_IN, D_HID), lambda i: (0, 0)),
            pl.BlockSpec((NC * NS, _ROW_BLK), lambda i: (0, i)),
            pl.BlockSpec((NC * NS, _ROW_BLK), lambda i: (0, i)),
        ],
        out_specs=(
            pl.BlockSpec((_ROW_BLK, D_HID // 2), lambda i: (i, 0)),
            pl.BlockSpec((_ROW_BLK, D_HID // 2), lambda i: (i, 0)),
            pl.BlockSpec((_ROW_BLK, 1), lambda i: (i, 0)),
            pl.BlockSpec((_ROW_BLK, 1), lambda i: (i, 0)),
        ),
        out_shape=(
            jax.ShapeDtypeStruct((N_PAD, D_HID // 2), jnp.bfloat16),
            jax.ShapeDtypeStruct((N_PAD, D_HID // 2), jnp.bfloat16),
            jax.ShapeDtypeStruct((N_PAD, 1), jnp.float32),
            jax.ShapeDtypeStruct((N_PAD, 1), jnp.float32),
        ),
    )(x_pad, w1, odeg_p, ideg_p)


def _mid_body(a1a_ref, a1b_ref, sin_ref, sout_ref, b1_ref, w2_ref,
              h2a_ref, h2b_ref):
    agg = jnp.concatenate(
        [a1a_ref[...].astype(jnp.float32), a1b_ref[...].astype(jnp.float32)],
        axis=1)
    z = jnp.maximum(agg * sin_ref[...] + b1_ref[...], 0.0)
    h2 = jnp.dot((z * sout_ref[...]).astype(jnp.bfloat16),
                 w2_ref[...].astype(jnp.bfloat16),
                 preferred_element_type=jnp.float32).astype(jnp.bfloat16)
    h2a_ref[...] = h2[:, :N_CLASSES // 2]
    h2b_ref[...] = h2[:, N_CLASSES // 2:]


def _mid_call(a1a, a1b, s_in, s_out, b1, w2):
    return pl.pallas_call(
        _mid_body,
        grid=(_N_BLKS,),
        in_specs=[
            pl.BlockSpec((_ROW_BLK, D_HID // 2), lambda i: (i, 0)),
            pl.BlockSpec((_ROW_BLK, D_HID // 2), lambda i: (i, 0)),
            pl.BlockSpec((_ROW_BLK, 1), lambda i: (i, 0)),
            pl.BlockSpec((_ROW_BLK, 1), lambda i: (i, 0)),
            pl.BlockSpec((1, D_HID), lambda i: (0, 0)),
            pl.BlockSpec((D_HID, N_CLASSES), lambda i: (0, 0)),
        ],
        out_specs=(
            pl.BlockSpec((_ROW_BLK, N_CLASSES // 2), lambda i: (i, 0)),
            pl.BlockSpec((_ROW_BLK, N_CLASSES // 2), lambda i: (i, 0)),
        ),
        out_shape=(
            jax.ShapeDtypeStruct((N_PAD, N_CLASSES // 2), jnp.bfloat16),
            jax.ShapeDtypeStruct((N_PAD, N_CLASSES // 2), jnp.bfloat16),
        ),
    )(a1a, a1b, s_in, s_out, b1, w2)


_F_BLK = 1000
_F_BLKS = N_NODES // _F_BLK


def _final_body(a2a_ref, a2b_ref, sin_ref, b2_ref, out_ref):
    agg = jnp.concatenate(
        [a2a_ref[...].astype(jnp.float32), a2b_ref[...].astype(jnp.float32)],
        axis=1)
    out_ref[...] = agg * sin_ref[...] + b2_ref[...]


def _final_call(a2a, a2b, s_in, b2):
    return pl.pallas_call(
        _final_body,
        grid=(_F_BLKS,),
        in_specs=[
            pl.BlockSpec((_F_BLK, N_CLASSES // 2), lambda i: (i, 0)),
            pl.BlockSpec((_F_BLK, N_CLASSES // 2), lambda i: (i, 0)),
            pl.BlockSpec((_F_BLK, 1), lambda i: (i, 0)),
            pl.BlockSpec((1, N_CLASSES), lambda i: (0, 0)),
        ],
        out_specs=pl.BlockSpec((_F_BLK, N_CLASSES), lambda i: (i, 0)),
        out_shape=jax.ShapeDtypeStruct((N_NODES, N_CLASSES), jnp.float32),
    )(a2a, a2b, s_in, b2)


def kernel(x, edge_index, W1, b1, W2, b2):
    src = edge_index[0]
    dst = edge_index[1]
    pad = jnp.full((E_PAD - N_EDGES,), N_NODES, dtype=jnp.int32)
    srcp = jnp.concatenate([src, pad])
    dstp = jnp.concatenate([dst, pad])
    src_b = srcp.reshape(NS, NB, B_EDGE)
    dst_b = dstp.reshape(NS, NB, B_EDGE)

    x_pad = jnp.pad(x, ((0, N_PAD - N_NODES), (0, 0)))
    z128 = jnp.zeros((ROWS_PER_TILE, 128), jnp.bfloat16)
    z64 = jnp.zeros((ROWS_PER_TILE, 64), jnp.bfloat16)

    odeg_p, ideg_p = _deg_kernel(srcp, dstp)

    h1a, h1b, s_in, s_out = _prep_call(x_pad, W1, odeg_p, ideg_p)
    a1a, a1b = _agg128(h1a, h1b, src_b, dst_b, z128)
    h2a, h2b = _mid_call(a1a, a1b, s_in, s_out, b1.reshape(1, -1), W2)
    a2a, a2b = _agg64(h2a, h2b, src_b, dst_b, z64)
    return _final_call(a2a, a2b, s_in, b2.reshape(1, -1))


# speedup vs baseline: 1.4862x; 1.1022x over previous
import functools

import jax
import jax.numpy as jnp
from jax import lax
from jax.experimental import pallas as pl
from jax.experimental.pallas import tpu as pltpu
from jax.experimental.pallas import tpu_sc as plsc

N_NODES = 10000
N_EDGES = 160000
D_IN = 256
D_HID = 256
N_CLASSES = 128

NC = 2
NS = 16
L = 16

N_PAD = 10240
ROWS_PER_TILE = N_PAD // NS
B_EDGE = 128
NB = 79
E_PAD = NS * NB * B_EDGE
EPT32 = E_PAD // (NC * NS)

_mesh = plsc.VectorSubcoreMesh(core_axis_name="c", subcore_axis_name="s")
_sc_params = pltpu.CompilerParams(needs_layout_passes=False,
                                  use_tc_tiling_on_sc=False)


@functools.partial(
    pl.kernel,
    out_type=(
        jax.ShapeDtypeStruct((NC * NS, N_PAD), jnp.float32),
        jax.ShapeDtypeStruct((NC * NS, N_PAD), jnp.float32),
    ),
    mesh=_mesh,
    scratch_types=[
        pltpu.VMEM((EPT32,), jnp.int32),
        pltpu.VMEM((EPT32,), jnp.int32),
        pltpu.VMEM((N_PAD,), jnp.float32),
        pltpu.VMEM((N_PAD,), jnp.float32),
    ],
    compiler_params=_sc_params,
)
def _deg_kernel(src_hbm, dst_hbm, odeg_hbm, ideg_hbm, src_v, dst_v, oacc, iacc):
    wid = lax.axis_index("s") * NC + lax.axis_index("c")
    base = wid * EPT32
    pltpu.sync_copy(src_hbm.at[pl.ds(base, EPT32)], src_v)
    pltpu.sync_copy(dst_hbm.at[pl.ds(base, EPT32)], dst_v)

    zeros16 = jnp.zeros((L,), jnp.float32)

    def zero_body(i, _):
        oacc[pl.ds(i * L, L)] = zeros16
        iacc[pl.ds(i * L, L)] = zeros16
        return 0

    lax.fori_loop(0, N_PAD // L, zero_body, 0)

    ones16 = jnp.ones((L,), jnp.float32)

    def edge_body(i, _):
        s16 = src_v[pl.ds(i * L, L)]
        d16 = dst_v[pl.ds(i * L, L)]
        plsc.addupdate_scatter(oacc, [s16], ones16)
        plsc.addupdate_scatter(iacc, [d16], ones16)
        return 0

    lax.fori_loop(0, EPT32 // L, edge_body, 0)

    pltpu.sync_copy(oacc, odeg_hbm.at[wid])
    pltpu.sync_copy(iacc, ideg_hbm.at[wid])


def _make_agg_kernel(dc):
    @functools.partial(
        pl.kernel,
        out_type=(
            jax.ShapeDtypeStruct((N_PAD, dc), jnp.bfloat16),
            jax.ShapeDtypeStruct((N_PAD, dc), jnp.bfloat16),
        ),
        mesh=_mesh,
        scratch_types=[
            pltpu.VMEM((NB, B_EDGE), jnp.int32),
            pltpu.VMEM((NB, B_EDGE), jnp.int32),
            pltpu.VMEM((3, B_EDGE, dc), jnp.bfloat16),
            pltpu.VMEM_SHARED((N_PAD, dc), jnp.bfloat16),
            pltpu.SemaphoreType.DMA,
            pltpu.SemaphoreType.DMA,
        ],
        compiler_params=_sc_params,
    )
    def agg(ha_hbm, hb_hbm, src_hbm, dst_hbm, zeros_hbm, outa_hbm, outb_hbm,
            src_v, dst_v, rows_v, acc_sh, gsem, ssem):
        cid = lax.axis_index("c")
        sid = lax.axis_index("s")
        row0 = sid * ROWS_PER_TILE

        pltpu.sync_copy(src_hbm.at[sid], src_v)
        pltpu.sync_copy(dst_hbm.at[sid], dst_v)
        pltpu.sync_copy(zeros_hbm, acc_sh.at[pl.ds(row0, ROWS_PER_TILE)])
        plsc.subcore_barrier()

        def run(h_hbm, nb):
            pltpu.async_copy(h_hbm.at[src_v.at[0]], rows_v.at[0], gsem)
            pltpu.async_copy(h_hbm.at[src_v.at[1]], rows_v.at[1], gsem)

            def body(j, _):
                slot = lax.rem(j, 3)
                pltpu.make_async_copy(h_hbm.at[src_v.at[j]], rows_v.at[slot],
                                      gsem).wait()

                @pl.when(j >= 1)
                def _():
                    jp = j - 1
                    pltpu.make_async_copy(rows_v.at[lax.rem(jp, 3)],
                                          acc_sh.at[dst_v.at[jp]], ssem).wait()

                @pl.when(j + 2 < nb)
                def _():
                    pltpu.async_copy(h_hbm.at[src_v.at[j + 2]],
                                     rows_v.at[lax.rem(j + 2, 3)], gsem)

                pltpu.async_copy(rows_v.at[slot], acc_sh.at[dst_v.at[j]],
                                 ssem, add=True)
                return 0

            lax.fori_loop(0, nb, body, 0)
            jl = nb - 1
            pltpu.make_async_copy(rows_v.at[lax.rem(jl, 3)],
                                  acc_sh.at[dst_v.at[jl]], ssem).wait()

        @pl.when(cid == 0)
        def _():
            run(ha_hbm, NB)

        @pl.when(cid == 1)
        def _():
            run(hb_hbm, NB)

        plsc.subcore_barrier()

        @pl.when(cid == 0)
        def _():
            pltpu.sync_copy(acc_sh.at[pl.ds(row0, ROWS_PER_TILE)],
                            outa_hbm.at[pl.ds(row0, ROWS_PER_TILE)])

        @pl.when(cid == 1)
        def _():
            pltpu.sync_copy(acc_sh.at[pl.ds(row0, ROWS_PER_TILE)],
                            outb_hbm.at[pl.ds(row0, ROWS_PER_TILE)])

    return agg


_agg128 = _make_agg_kernel(128)
_agg64 = _make_agg_kernel(64)


_ROW_BLK = 512
_N_BLKS = N_PAD // _ROW_BLK


def _scales(odeg_blk, ideg_blk):
    odeg = jnp.sum(odeg_blk, axis=0)
    ideg = jnp.sum(ideg_blk, axis=0)
    return (lax.rsqrt(jnp.maximum(odeg, 1.0)), lax.rsqrt(jnp.maximum(ideg, 1.0)))


def _prep_body(x_ref, w1_ref, odeg_ref, ideg_ref, h1a_ref, h1b_ref,
               sin_ref, sout_ref):
    s_out, s_in = _scales(odeg_ref[...], ideg_ref[...])
    sin_ref[...] = s_in[:, None]
    sout_ref[...] = s_out[:, None]
    h = jnp.dot((x_ref[...] * s_out[:, None]).astype(jnp.bfloat16),
                w1_ref[...].astype(jnp.bfloat16),
                preferred_element_type=jnp.float32).astype(jnp.bfloat16)
    h1a_ref[...] = h[:, :D_HID // 2]
    h1b_ref[...] = h[:, D_HID // 2:]


def _prep_call(x_pad, w1, odeg_p, ideg_p):
    return pl.pallas_call(
        _prep_body,
        grid=(_N_BLKS,),
        in_specs=[
            pl.BlockSpec((_ROW_BLK, D_IN), lambda i: (i, 0)),
            pl.BlockSpec((D_IN, D_HID), lambda i: (0, 0)),
            pl.BlockSpec((NC * NS, _ROW_BLK), lambda i: (0, i)),
            pl.BlockSpec((NC * NS, _ROW_BLK), lambda i: (0, i)),
        ],
        out_specs=(
            pl.BlockSpec((_ROW_BLK, D_HID // 2), lambda i: (i, 0)),
            pl.BlockSpec((_ROW_BLK, D_HID // 2), lambda i: (i, 0)),
            pl.BlockSpec((_ROW_BLK, 1), lambda i: (i, 0)),
            pl.BlockSpec((_ROW_BLK, 1), lambda i: (i, 0)),
        ),
        out_shape=(
            jax.ShapeDtypeStruct((N_PAD, D_HID // 2), jnp.bfloat16),
            jax.ShapeDtypeStruct((N_PAD, D_HID // 2), jnp.bfloat16),
            jax.ShapeDtypeStruct((N_PAD, 1), jnp.float32),
            jax.ShapeDtypeStruct((N_PAD, 1), jnp.float32),
        ),
    )(x_pad, w1, odeg_p, ideg_p)


def _mid_body(a1a_ref, a1b_ref, sin_ref, sout_ref, b1_ref, w2_ref,
              h2a_ref, h2b_ref):
    agg = jnp.concatenate(
        [a1a_ref[...].astype(jnp.float32), a1b_ref[...].astype(jnp.float32)],
        axis=1)
    z = jnp.maximum(agg * sin_ref[...] + b1_ref[...], 0.0)
    h2 = jnp.dot((z * sout_ref[...]).astype(jnp.bfloat16),
                 w2_ref[...].astype(jnp.bfloat16),
                 preferred_element_type=jnp.float32).astype(jnp.bfloat16)
    h2a_ref[...] = h2[:, :N_CLASSES // 2]
    h2b_ref[...] = h2[:, N_CLASSES // 2:]


def _mid_call(a1a, a1b, s_in, s_out, b1, w2):
    return pl.pallas_call(
        _mid_body,
        grid=(_N_BLKS,),
        in_specs=[
            pl.BlockSpec((_ROW_BLK, D_HID // 2), lambda i: (i, 0)),
            pl.BlockSpec((_ROW_BLK, D_HID // 2), lambda i: (i, 0)),
            pl.BlockSpec((_ROW_BLK, 1), lambda i: (i, 0)),
            pl.BlockSpec((_ROW_BLK, 1), lambda i: (i, 0)),
            pl.BlockSpec((1, D_HID), lambda i: (0, 0)),
            pl.BlockSpec((D_HID, N_CLASSES), lambda i: (0, 0)),
        ],
        out_specs=(
            pl.BlockSpec((_ROW_BLK, N_CLASSES // 2), lambda i: (i, 0)),
            pl.BlockSpec((_ROW_BLK, N_CLASSES // 2), lambda i: (i, 0)),
        ),
        out_shape=(
            jax.ShapeDtypeStruct((N_PAD, N_CLASSES // 2), jnp.bfloat16),
            jax.ShapeDtypeStruct((N_PAD, N_CLASSES // 2), jnp.bfloat16),
        ),
    )(a1a, a1b, s_in, s_out, b1, w2)


_F_BLK = 1000
_F_BLKS = N_NODES // _F_BLK


def _final_body(a2a_ref, a2b_ref, sin_ref, b2_ref, out_ref):
    agg = jnp.concatenate(
        [a2a_ref[...].astype(jnp.float32), a2b_ref[...].astype(jnp.float32)],
        axis=1)
    out_ref[...] = agg * sin_ref[...] + b2_ref[...]


def _final_call(a2a, a2b, s_in, b2):
    return pl.pallas_call(
        _final_body,
        grid=(_F_BLKS,),
        in_specs=[
            pl.BlockSpec((_F_BLK, N_CLASSES // 2), lambda i: (i, 0)),
            pl.BlockSpec((_F_BLK, N_CLASSES // 2), lambda i: (i, 0)),
            pl.BlockSpec((_F_BLK, 1), lambda i: (i, 0)),
            pl.BlockSpec((1, N_CLASSES), lambda i: (0, 0)),
        ],
        out_specs=pl.BlockSpec((_F_BLK, N_CLASSES), lambda i: (i, 0)),
        out_shape=jax.ShapeDtypeStruct((N_NODES, N_CLASSES), jnp.float32),
    )(a2a, a2b, s_in, b2)


def kernel(x, edge_index, W1, b1, W2, b2):
    src = edge_index[0]
    dst = edge_index[1]
    pad = jnp.full((E_PAD - N_EDGES,), N_NODES, dtype=jnp.int32)
    srcp = jnp.concatenate([src, pad])
    dstp = jnp.concatenate([dst, pad])
    src_b = srcp.reshape(NS, NB, B_EDGE)
    dst_b = dstp.reshape(NS, NB, B_EDGE)

    x_pad = jnp.pad(x, ((0, N_PAD - N_NODES), (0, 0)))
    z128 = jnp.zeros((ROWS_PER_TILE, 128), jnp.bfloat16)
    z64 = jnp.zeros((ROWS_PER_TILE, 64), jnp.bfloat16)

    odeg_p, ideg_p = _deg_kernel(srcp, dstp)

    h1a, h1b, s_in, s_out = _prep_call(x_pad, W1, odeg_p, ideg_p)
    a1a, a1b = _agg128(h1a, h1b, src_b, dst_b, z128)
    h2a, h2b = _mid_call(a1a, a1b, s_in, s_out, b1.reshape(1, -1), W2)
    a2a, a2b = _agg64(h2a, h2b, src_b, dst_b, z64)
    return _final_call(a2a, a2b, s_in, b2.reshape(1, -1))
